# baseline (speedup 1.0000x reference)
import functools
import jax
import jax.numpy as jnp
from jax import lax
from jax.experimental import pallas as pl
from jax.experimental.pallas import tpu as pltpu
from jax.experimental.pallas import tpu_sc as plsc

VOCAB = 1000000
D = 64
SCALE = 8.0

_info = plsc.get_sparse_core_info()
NC = _info.num_cores
NS = _info.num_subcores
L = _info.num_lanes
NW = NC * NS

XROWS = 4096
XCOLS = 200
R_PER_W = XROWS // NW
CH0, CH1 = 96, 104
NBUF = 2
NOUTER = R_PER_W // NBUF
RU = 4

_mesh = plsc.VectorSubcoreMesh(core_axis_name="c", subcore_axis_name="s")


@functools.partial(
    pl.kernel,
    mesh=_mesh,
    compiler_params=pltpu.CompilerParams(use_tc_tiling_on_sc=False),
    out_type=jax.ShapeDtypeStruct((XROWS, XCOLS, D), jnp.float32),
    scratch_types=[
        pltpu.VMEM((R_PER_W, XCOLS), jnp.int32),
        pltpu.VMEM((NBUF, XCOLS, D), jnp.float32),
        pltpu.VMEM((NBUF, XCOLS, D), jnp.float32),
        pltpu.SemaphoreType.DMA((NBUF,)),
        pltpu.SemaphoreType.DMA((NBUF,)),
    ],
)
def _embed_kernel(x_hbm, lut_hbm, out_hbm, idx_v, gbuf, obuf, gsem, psem):
    wid = lax.axis_index("s") * NC + lax.axis_index("c")
    row0 = wid * R_PER_W
    pltpu.sync_copy(x_hbm.at[pl.ds(row0, R_PER_W)], idx_v)

    def gather_copies(r, b):
        return (
            pltpu.make_async_copy(
                lut_hbm.at[idx_v.at[r, pl.ds(0, CH0)]],
                gbuf.at[b, pl.ds(0, CH0)], gsem.at[b]),
            pltpu.make_async_copy(
                lut_hbm.at[idx_v.at[r, pl.ds(CH0, CH1)]],
                gbuf.at[b, pl.ds(CH0, CH1)], gsem.at[b]),
        )

    def put_copy(r, b):
        return pltpu.make_async_copy(obuf.at[b], out_hbm.at[row0 + r], psem.at[b])

    def start_gathers(r, b):
        for c in gather_copies(r, b):
            c.start()

    def wait_gathers(r, b):
        for c in gather_copies(r, b):
            c.wait()

    def scale_chunk(b):
        def mrow(i, c):
            for u in range(RU):
                ii = i * RU + u
                for q in range(D // L):
                    sl = pl.ds(q * L, L)
                    obuf[b, ii, sl] = gbuf[b, ii, sl] * SCALE
            return c
        lax.fori_loop(0, XCOLS // RU, mrow, 0)

    def step(r, b, first, last):
        wait_gathers(r, b)
        if not first:
            put_copy(r, b).wait()
        scale_chunk(b)
        if not last:
            start_gathers(r + NBUF, b)
        put_copy(r, b).start()

    for b in range(NBUF):
        start_gathers(b, b)

    for b in range(NBUF):
        step(b, b, True, False)

    def outer(g, c):
        for b in range(NBUF):
            step(g * NBUF + b, b, False, False)
        return c

    lax.fori_loop(1, NOUTER - 1, outer, 0)

    for b in range(NBUF):
        step((NOUTER - 1) * NBUF + b, b, False, True)

    for b in range(NBUF):
        put_copy((NOUTER - 1) * NBUF + b, b).wait()


def kernel(x, lut):
    return _embed_kernel(x.astype(jnp.int32), lut)


# speedup vs baseline: 1.2267x; 1.2267x over previous
import functools
import jax
import jax.numpy as jnp
from jax import lax
from jax.experimental import pallas as pl
from jax.experimental.pallas import tpu as pltpu
from jax.experimental.pallas import tpu_sc as plsc

VOCAB = 1000000
D = 64
DPAD = 128
SCALE = 8.0

_info = plsc.get_sparse_core_info()
NC = _info.num_cores
NS = _info.num_subcores
L = _info.num_lanes
NW = NC * NS

XROWS = 4096
XCOLS = 200
B = XROWS * XCOLS
B_PER_W = B // NW
CH = 128
NCHUNK = B_PER_W // CH
NBUF = 2
NOUTER = NCHUNK // NBUF
RU = 4

_mesh = plsc.VectorSubcoreMesh(core_axis_name="c", subcore_axis_name="s")


@functools.partial(
    pl.kernel,
    mesh=_mesh,
    compiler_params=pltpu.CompilerParams(use_tc_tiling_on_sc=True),
    out_type=jax.ShapeDtypeStruct((B, D), jnp.float32),
    scratch_types=[
        pltpu.VMEM((B_PER_W,), jnp.int32),
        pltpu.VMEM((NBUF, CH, DPAD), jnp.float32),
        pltpu.VMEM((NBUF, CH, D), jnp.float32),
        pltpu.SemaphoreType.DMA((NBUF,)),
        pltpu.SemaphoreType.DMA((NBUF,)),
    ],
)
def _embed_kernel(x_hbm, lut_hbm, out_hbm, idx_v, gbuf, obuf, gsem, psem):
    wid = lax.axis_index("s") * NC + lax.axis_index("c")
    base = wid * B_PER_W
    pltpu.sync_copy(x_hbm.at[pl.ds(base, B_PER_W)], idx_v)

    def gather_copy(j, b):
        return pltpu.make_async_copy(
            lut_hbm.at[idx_v.at[pl.ds(j * CH, CH)]], gbuf.at[b], gsem.at[b])

    def put_copy(j, b):
        return pltpu.make_async_copy(
            obuf.at[b], out_hbm.at[pl.ds(base + j * CH, CH)], psem.at[b])

    def scale_chunk(b):
        def mrow(i, c):
            for u in range(RU):
                ii = i * RU + u
                for q in range(D // L):
                    obuf[b, ii, pl.ds(q * L, L)] = (
                        gbuf[b, ii, pl.ds(q * L, L)] * SCALE)
            return c
        lax.fori_loop(0, CH // RU, mrow, 0)

    def step(j, b, first, last):
        gather_copy(j, b).wait()
        if not first:
            put_copy(j, b).wait()
        scale_chunk(b)
        if not last:
            gather_copy(j + NBUF, b).start()
        put_copy(j, b).start()

    for b in range(NBUF):
        gather_copy(b, b).start()

    for b in range(NBUF):
        step(b, b, True, False)

    def outer(g, c):
        for b in range(NBUF):
            step(g * NBUF + b, b, False, False)
        return c

    lax.fori_loop(1, NOUTER - 1, outer, 0)

    for b in range(NBUF):
        step((NOUTER - 1) * NBUF + b, b, False, True)

    for b in range(NBUF):
        put_copy((NOUTER - 1) * NBUF + b, b).wait()


def kernel(x, lut):
    lutp = jnp.pad(lut, ((0, 0), (0, DPAD - D)))
    out = _embed_kernel(x.astype(jnp.int32).reshape(-1), lutp)
    return out.reshape(XROWS, XCOLS, D)
